# baseline (speedup 1.0000x reference)
import jax
import jax.numpy as jnp
from jax.experimental import pallas as pl

_BB = 64


def _body(x_ref, pos_ref, out_ref):
    xb = x_ref[...]
    pos = pos_ref[...]
    bb, l = xb.shape
    d = pos.shape[-1]
    lane = jax.lax.broadcasted_iota(jnp.int32, (bb, l, d), 2)
    onehot = (lane == xb[:, :, None]).astype(jnp.float32)
    out_ref[...] = onehot + pos[None, :, :]


def kernel(x, pos_table):
    B, L = x.shape
    D = pos_table.shape[-1]
    x = x.astype(jnp.int32)
    return pl.pallas_call(
        _body,
        grid=(B // _BB,),
        in_specs=[
            pl.BlockSpec((_BB, L), lambda i: (i, 0)),
            pl.BlockSpec((L, D), lambda i: (0, 0)),
        ],
        out_specs=pl.BlockSpec((_BB, L, D), lambda i: (i, 0, 0)),
        out_shape=jax.ShapeDtypeStruct((B, L, D), jnp.float32),
    )(x, pos_table)


# speedup vs baseline: 1.0724x; 1.0724x over previous
import jax
import jax.numpy as jnp
from jax.experimental import pallas as pl
from jax.experimental.pallas import tpu as pltpu

_BB = 128


def _body(x_ref, pos_ref, out_ref):
    xb = x_ref[...]
    pos = pos_ref[...]
    bb, l = xb.shape
    d = pos.shape[-1]
    pos1 = pos + 1.0
    lane = jax.lax.broadcasted_iota(jnp.int32, (bb, l, d), 2)
    eq = lane == xb[:, :, None]
    out_ref[...] = jnp.where(eq, pos1[None, :, :], pos[None, :, :])


def kernel(x, pos_table):
    B, L = x.shape
    D = pos_table.shape[-1]
    x = x.astype(jnp.int32)
    return pl.pallas_call(
        _body,
        grid=(B // _BB,),
        in_specs=[
            pl.BlockSpec((_BB, L), lambda i: (i, 0)),
            pl.BlockSpec((L, D), lambda i: (0, 0)),
        ],
        out_specs=pl.BlockSpec((_BB, L, D), lambda i: (i, 0, 0)),
        out_shape=jax.ShapeDtypeStruct((B, L, D), jnp.float32),
        compiler_params=pltpu.CompilerParams(
            dimension_semantics=("arbitrary",)),
    )(x, pos_table)
